# baseline (speedup 1.0000x reference)
import functools

import jax
import jax.numpy as jnp
from jax import lax
from jax.experimental import pallas as pl
from jax.experimental.pallas import tpu as pltpu
from jax.experimental.pallas import tpu_sc as plsc

N = 10000
E = 320000
D_NODE = 128
D_EDGE = 16
HIDDEN = 64

NC = 2
NS = 16
SW = 128
GW = 2
PW = 4

_f32 = jnp.float32


def _sds(shape, dtype=_f32):
    return jax.ShapeDtypeStruct(shape, dtype)


def _tc_pre(x, W_msg, w1):
    def k(x_ref, wm_ref, w1_ref, xa_ref, xb_ref):
        w1a = w1_ref[0:D_NODE, :]
        w1b = w1_ref[D_NODE:2 * D_NODE, :]
        w1c = w1_ref[2 * D_NODE:2 * D_NODE + D_EDGE, :]
        A = w1a + jnp.dot(wm_ref[...], w1c, preferred_element_type=_f32)
        xa_ref[...] = jnp.dot(x_ref[...], A, preferred_element_type=_f32)
        xb_ref[...] = jnp.dot(x_ref[...], w1b, preferred_element_type=_f32)

    return pl.pallas_call(
        k,
        out_shape=(_sds((N, HIDDEN)), _sds((N, HIDDEN))),
    )(x, W_msg, w1)


def _sc_gather(xa, xb, src2d, dst2d, zeros_nk, ones_wk):
    mesh = plsc.VectorSubcoreMesh(core_axis_name="c", subcore_axis_name="s")
    ROWS = N // NS

    @functools.partial(
        pl.kernel,
        mesh=mesh,
        out_type=(_sds((E, HIDDEN)), _sds((E, HIDDEN)),
                  _sds((NC * N, D_EDGE))),
        scratch_types=[
            pltpu.VMEM((SW, D_EDGE), _f32),
            pltpu.VMEM_SHARED((N, D_EDGE), _f32),
            pltpu.SemaphoreType.DMA,
            pltpu.SemaphoreType.DMA,
        ],
        compiler_params=pltpu.CompilerParams(use_tc_tiling_on_sc=False),
    )
    def k(xa_hbm, xb_hbm, src_hbm, dst_hbm, z_hbm, ones_hbm,
          g1_hbm, g2_hbm, cnt_hbm, ones_v, cnt_sh, sem1, sem2):
        cid = lax.axis_index("c")
        sid = lax.axis_index("s")
        r0 = sid * ROWS
        pltpu.sync_copy(z_hbm.at[pl.ds(r0, ROWS)], cnt_sh.at[pl.ds(r0, ROWS)])
        pltpu.sync_copy(ones_hbm, ones_v)
        plsc.subcore_barrier()

        def body(isrc_v, idst_v, o1_v, o2_v):
            cps = []
            for j in range(GW):
                blk = pl.ds(j * SW, SW)
                cps.append(pltpu.async_copy(
                    xa_hbm.at[isrc_v.at[j]], o1_v.at[blk], sem1))
                cps.append(pltpu.async_copy(
                    xb_hbm.at[idst_v.at[j]], o2_v.at[blk], sem2))
            for j in range(GW):
                pltpu.sync_copy(ones_v, cnt_sh.at[idst_v.at[j]], add=True)
            for cp in cps:
                cp.wait()

        pltpu.emit_pipeline(
            body,
            grid=(E // (GW * SW),),
            in_specs=[
                pl.BlockSpec((GW, SW), lambda i: (i, 0)),
                pl.BlockSpec((GW, SW), lambda i: (i, 0)),
            ],
            out_specs=[
                pl.BlockSpec((GW * SW, HIDDEN), lambda i: (i, 0)),
                pl.BlockSpec((GW * SW, HIDDEN), lambda i: (i, 0)),
            ],
            core_axis_name=("c", "s"),
            dimension_semantics=(pltpu.PARALLEL,),
        )(src_hbm, dst_hbm, g1_hbm, g2_hbm)

        plsc.subcore_barrier()
        off = cid * N + r0
        pltpu.sync_copy(cnt_sh.at[pl.ds(r0, ROWS)], cnt_hbm.at[pl.ds(off, ROWS)])

    return k(xa, xb, src2d, dst2d, zeros_nk, ones_wk)


def _tc_edge(g1, g2, e, w1c, b1r, w2, b2r):
    BE = 8000

    def k(g1_ref, g2_ref, e_ref, wc_ref, b1_ref, w2_ref, b2_ref, o_ref):
        pre = (g1_ref[...] + g2_ref[...]
               + jnp.dot(e_ref[...], wc_ref[...], preferred_element_type=_f32)
               + b1_ref[...])
        h = jnp.maximum(pre, 0.0)
        o_ref[...] = e_ref[...] + jnp.dot(h, w2_ref[...],
                                          preferred_element_type=_f32) + b2_ref[...]

    return pl.pallas_call(
        k,
        grid=(E // BE,),
        in_specs=[
            pl.BlockSpec((BE, HIDDEN), lambda i: (i, 0)),
            pl.BlockSpec((BE, HIDDEN), lambda i: (i, 0)),
            pl.BlockSpec((BE, D_EDGE), lambda i: (i, 0)),
            pl.BlockSpec((D_EDGE, HIDDEN), lambda i: (0, 0)),
            pl.BlockSpec((1, HIDDEN), lambda i: (0, 0)),
            pl.BlockSpec((HIDDEN, D_EDGE), lambda i: (0, 0)),
            pl.BlockSpec((1, D_EDGE), lambda i: (0, 0)),
        ],
        out_specs=pl.BlockSpec((BE, D_EDGE), lambda i: (i, 0)),
        out_shape=_sds((E, D_EDGE)),
    )(g1, g2, e, w1c, b1r, w2, b2r)


def _sc_scatter(e_new, dst2d, zeros_nk):
    mesh = plsc.VectorSubcoreMesh(core_axis_name="c", subcore_axis_name="s")
    ROWS = N // NS

    @functools.partial(
        pl.kernel,
        mesh=mesh,
        out_type=_sds((NC * N, D_EDGE)),
        scratch_types=[
            pltpu.VMEM_SHARED((N, D_EDGE), _f32),
        ],
        compiler_params=pltpu.CompilerParams(use_tc_tiling_on_sc=False),
    )
    def k(enew_hbm, dst_hbm, z_hbm, sums_hbm, sums_sh):
        cid = lax.axis_index("c")
        sid = lax.axis_index("s")
        r0 = sid * ROWS
        pltpu.sync_copy(z_hbm.at[pl.ds(r0, ROWS)], sums_sh.at[pl.ds(r0, ROWS)])
        plsc.subcore_barrier()

        def body(e_v, i_v):
            for j in range(PW):
                pltpu.sync_copy(e_v.at[pl.ds(j * SW, SW)],
                                sums_sh.at[i_v.at[j]], add=True)

        pltpu.emit_pipeline(
            body,
            grid=(E // (PW * SW),),
            in_specs=[
                pl.BlockSpec((PW * SW, D_EDGE), lambda i: (i, 0)),
                pl.BlockSpec((PW, SW), lambda i: (i, 0)),
            ],
            out_specs=[],
            core_axis_name=("c", "s"),
            dimension_semantics=(pltpu.PARALLEL,),
        )(enew_hbm, dst_hbm)

        plsc.subcore_barrier()
        off = cid * N + r0
        pltpu.sync_copy(sums_sh.at[pl.ds(r0, ROWS)], sums_hbm.at[pl.ds(off, ROWS)])

    return k(e_new, dst2d, zeros_nk)


def _tc_node(x, sums_p, cnt_p, v1a, v1b, vb1r, v2, vb2r):
    def k(x_ref, s_ref, c_ref, v1a_ref, v1b_ref, vb1_ref, v2_ref, vb2_ref,
          o_ref):
        s = s_ref[0:N, :] + s_ref[N:2 * N, :]
        cnt = c_ref[0:N, 0:1] + c_ref[N:2 * N, 0:1]
        m = s / jnp.maximum(cnt, 1.0)
        pre = (jnp.dot(x_ref[...], v1a_ref[...], preferred_element_type=_f32)
               + jnp.dot(m, v1b_ref[...], preferred_element_type=_f32)
               + vb1_ref[...])
        hv = jnp.maximum(pre, 0.0)
        o_ref[...] = x_ref[...] + jnp.dot(hv, v2_ref[...],
                                          preferred_element_type=_f32) + vb2_ref[...]

    return pl.pallas_call(
        k,
        out_shape=_sds((N, D_NODE)),
    )(x, sums_p, cnt_p, v1a, v1b, vb1r, v2, vb2r)


def kernel(x, edge_index, e, W_msg, w1, b1, w2, b2, v1, vb1, v2, vb2):
    src2d = edge_index[0].astype(jnp.int32).reshape(E // SW, SW)
    dst2d = edge_index[1].astype(jnp.int32).reshape(E // SW, SW)
    zeros_nk = jnp.zeros((N, D_EDGE), _f32)
    ones_wk = jnp.ones((SW, D_EDGE), _f32)

    xa, xb = _tc_pre(x, W_msg, w1)
    g1, g2, cnt_p = _sc_gather(xa, xb, src2d, dst2d, zeros_nk, ones_wk)

    w1c = w1[2 * D_NODE:2 * D_NODE + D_EDGE, :]
    e_new = _tc_edge(g1, g2, e, w1c, b1.reshape(1, HIDDEN), w2,
                     b2.reshape(1, D_EDGE))

    sums_p = _sc_scatter(e_new, dst2d, zeros_nk)

    x_new = _tc_node(x, sums_p, cnt_p, v1[0:D_NODE, :],
                     v1[D_NODE:D_NODE + D_EDGE, :],
                     vb1.reshape(1, HIDDEN), v2, vb2.reshape(1, D_NODE))
    return (x_new, e_new)
